# baseline (speedup 1.0000x reference)
import jax
import jax.numpy as jnp
from jax import lax
from jax.experimental import pallas as pl
from jax.experimental.pallas import tpu as pltpu
from jax.experimental.pallas import tpu_sc as plsc

_B, _MQ, _QL, _V, _E, _H, _DV = 16, 8, 16, 5000, 512, 512, 1024
_S = _MQ * _QL
_N = _S * _B
_VT = 640
_NVT = (_V + _VT - 1) // _VT


def _sc_gather(table, idx):
    n, d = idx.shape[0], table.shape[1]
    nw = 32
    per = n // nw
    mesh = plsc.VectorSubcoreMesh(core_axis_name="c", subcore_axis_name="s")

    def body(table_hbm, idx_hbm, out_hbm, idx_v, rows_v, sem):
        wid = lax.axis_index("s") * 2 + lax.axis_index("c")
        base = wid * per
        pltpu.sync_copy(idx_hbm.at[pl.ds(base, per)], idx_v)
        pltpu.async_copy(table_hbm.at[idx_v], rows_v, sem).wait()
        pltpu.sync_copy(rows_v, out_hbm.at[pl.ds(base, per)])

    return pl.kernel(
        body,
        out_type=jax.ShapeDtypeStruct((n, d), table.dtype),
        mesh=mesh,
        scratch_types=[
            pltpu.VMEM((per,), jnp.int32),
            pltpu.VMEM((per, d), table.dtype),
            pltpu.SemaphoreType.DMA,
        ],
    )(table, idx)


def _prep(Wih, Whh, vis, b2d):
    def body(wih_ref, whh_ref, v_ref, b_ref, wie_o, whh_o, vz_o):
        wih = wih_ref[...]
        wie_o[...] = wih[:, :_E].astype(jnp.bfloat16)
        whh_o[...] = whh_ref[...].astype(jnp.bfloat16)
        viszT = jnp.dot(wih[:, _E:], v_ref[...].T,
                        preferred_element_type=jnp.float32)
        vz_o[...] = viszT.T + b_ref[...]

    return pl.pallas_call(
        body,
        out_shape=(
            jax.ShapeDtypeStruct((4 * _H, _E), jnp.bfloat16),
            jax.ShapeDtypeStruct((4 * _H, _H), jnp.bfloat16),
            jax.ShapeDtypeStruct((_B, 4 * _H), jnp.float32),
        ),
    )(Wih, Whh, vis, b2d)


def _lstm_scan(xe3, wie16, whh16, visz, sel):
    def body(xe_ref, wie_ref, whh_ref, vz_ref, sel_ref, hs_ref,
             wieT_s, whhT_s, ch_ref, cc_ref):
        qi = pl.program_id(0)

        @pl.when(qi == 0)
        def _():
            wieT_s[...] = wie_ref[...].T
            whhT_s[...] = whh_ref[...].T
            ch_ref[...] = jnp.zeros_like(ch_ref)
            cc_ref[...] = jnp.zeros_like(cc_ref)

        xe = xe_ref[...].reshape(_QL * _B, _E).astype(jnp.bfloat16)
        z0 = jnp.dot(xe, wieT_s[...], preferred_element_type=jnp.float32)
        z0 = z0.reshape(_QL, _B, 4 * _H) + vz_ref[...][None]
        whhT = whhT_s[...]
        h = ch_ref[...]
        c = cc_ref[...]
        carry_h = h
        carry_c = c
        for t in range(_QL):
            z = z0[t] + jnp.dot(h.astype(jnp.bfloat16), whhT,
                                preferred_element_type=jnp.float32)
            zi = z[:, 0 * _H:1 * _H]
            zf = z[:, 1 * _H:2 * _H]
            zg = z[:, 2 * _H:3 * _H]
            zo = z[:, 3 * _H:4 * _H]
            c = jax.nn.sigmoid(zf) * c + jax.nn.sigmoid(zi) * jnp.tanh(zg)
            h = jax.nn.sigmoid(zo) * jnp.tanh(c)
            hs_ref[:, t, :] = h
            s = sel_ref[0, :, t:t + 1]
            carry_h = s * h + (1.0 - s) * carry_h
            carry_c = s * c + (1.0 - s) * carry_c
        ch_ref[...] = carry_h
        cc_ref[...] = carry_c

    return pl.pallas_call(
        body,
        grid=(_MQ,),
        in_specs=[
            pl.BlockSpec((_QL, _B, _E), lambda i: (i, 0, 0)),
            pl.BlockSpec((4 * _H, _E), lambda i: (0, 0)),
            pl.BlockSpec((4 * _H, _H), lambda i: (0, 0)),
            pl.BlockSpec((_B, 4 * _H), lambda i: (0, 0)),
            pl.BlockSpec((1, _B, _QL), lambda i: (i, 0, 0)),
        ],
        out_specs=pl.BlockSpec((_B, _QL, _H), lambda i: (0, i, 0)),
        out_shape=jax.ShapeDtypeStruct((_B, _S, _H), jnp.float32),
        scratch_shapes=[
            pltpu.VMEM((_E, 4 * _H), jnp.bfloat16),
            pltpu.VMEM((_H, 4 * _H), jnp.bfloat16),
            pltpu.VMEM((_B, _H), jnp.float32),
            pltpu.VMEM((_B, _H), jnp.float32),
        ],
    )(xe3, wie16, whh16, visz, sel)


def _proj(hs3, outW, outbc, vmaskr):
    def body(h_ref, w_ref, b_ref, m_ref, o_ref, hsT_s):
        j = pl.program_id(0)

        @pl.when(j == 0)
        def _():
            hs2 = h_ref[...].reshape(_N, _H)
            hsT_s[...] = hs2.T.astype(jnp.bfloat16)

        ow = w_ref[...].astype(jnp.bfloat16)
        acc = jnp.dot(ow, hsT_s[...], preferred_element_type=jnp.float32)
        o_ref[...] = (acc + b_ref[...]) * m_ref[...]

    return pl.pallas_call(
        body,
        grid=(_NVT,),
        in_specs=[
            pl.BlockSpec((_B, _S, _H), lambda j: (0, 0, 0)),
            pl.BlockSpec((_VT, _H), lambda j: (j, 0)),
            pl.BlockSpec((_VT, 1), lambda j: (j, 0)),
            pl.BlockSpec((1, _N), lambda j: (0, 0)),
        ],
        out_specs=pl.BlockSpec((_VT, _N), lambda j: (j, 0)),
        out_shape=jax.ShapeDtypeStruct((_V, _N), jnp.float32),
        scratch_shapes=[pltpu.VMEM((_H, _N), jnp.bfloat16)],
    )(hs3, outW, outbc, vmaskr)


def kernel(source_questions, question_lengths, visual_features, unrolled_dialogue,
           cumulative_lengths, num_questions, object_categories, object_bboxes,
           emb, Wih, Whh, b, outW, outb, g_emb, g_Wih, g_Whh, g_b,
           cat_emb, W1, b1, W2, b2):
    toks = source_questions.transpose(1, 2, 0).reshape(_N).astype(jnp.int32)
    xe = _sc_gather(emb, toks)
    xe3 = xe.reshape(_S, _B, _E)

    lens = question_lengths.astype(jnp.int32)
    nq = num_questions.astype(jnp.int32)
    running = jnp.arange(_MQ)[None, :] < nq[:, None]
    tix = jnp.arange(_QL)
    sel = (lens[:, :, None] - 1 == tix[None, None, :]) & running[:, :, None]
    sel = sel.transpose(1, 0, 2).astype(jnp.float32)
    valid = (tix[None, None, :] < lens[:, :, None]) & running[:, :, None]
    vmaskr = valid.reshape(1, _N).astype(jnp.float32)

    wie16, whh16, visz = _prep(Wih, Whh, visual_features,
                               b.reshape(1, 4 * _H))
    hs3 = _lstm_scan(xe3, wie16, whh16, visz, sel)
    outT = _proj(hs3, outW, outb.reshape(_V, 1), vmaskr)
    return outT.T


# speedup vs baseline: 1.0702x; 1.0702x over previous
import jax
import jax.numpy as jnp
from jax import lax
from jax.experimental import pallas as pl
from jax.experimental.pallas import tpu as pltpu
from jax.experimental.pallas import tpu_sc as plsc

_B, _MQ, _QL, _V, _E, _H, _DV = 16, 8, 16, 5000, 512, 512, 1024
_S = _MQ * _QL
_N = _S * _B
_VT = 640
_NVT = (_V + _VT - 1) // _VT


def _sc_gather(table, idx):
    n, d = idx.shape[0], table.shape[1]
    nw = 32
    per = n // nw
    mesh = plsc.VectorSubcoreMesh(core_axis_name="c", subcore_axis_name="s")

    def body(table_hbm, idx_hbm, out_hbm, idx_v, rows_v, sem):
        wid = lax.axis_index("s") * 2 + lax.axis_index("c")
        base = wid * per
        pltpu.sync_copy(idx_hbm.at[pl.ds(base, per)], idx_v)
        pltpu.async_copy(table_hbm.at[idx_v], rows_v, sem).wait()
        pltpu.sync_copy(rows_v, out_hbm.at[pl.ds(base, per)])

    return pl.kernel(
        body,
        out_type=jax.ShapeDtypeStruct((n, d), table.dtype),
        mesh=mesh,
        scratch_types=[
            pltpu.VMEM((per,), jnp.int32),
            pltpu.VMEM((per, d), table.dtype),
            pltpu.SemaphoreType.DMA,
        ],
    )(table, idx)


def _scan_proj(xe3, Wih, Whh, outW, visT, b2d, sel, vmaskr, outbc):
    def body(xe_ref, wih_ref, whh_ref, ow_ref, vt_ref, b_ref, sel_ref,
             vm_ref, ob_ref, out_ref,
             wieT_s, whhT_s, vz_s, hs_ref, hsT_s, ch_ref, cc_ref):
        i = pl.program_id(0)

        @pl.when(i == 0)
        def _():
            wih = wih_ref[...]
            wieT_s[...] = wih[:, :_E].T.astype(jnp.bfloat16)
            whhT_s[...] = whh_ref[...].T.astype(jnp.bfloat16)
            viszT = jnp.dot(wih[:, _E:], vt_ref[...],
                            preferred_element_type=jnp.float32)
            vz_s[...] = viszT.T + b_ref[...]
            ch_ref[...] = jnp.zeros_like(ch_ref)
            cc_ref[...] = jnp.zeros_like(cc_ref)

        @pl.when(i < _MQ)
        def _scan():
            xe = xe_ref[...].reshape(_QL * _B, _E).astype(jnp.bfloat16)
            z0 = jnp.dot(xe, wieT_s[...], preferred_element_type=jnp.float32)
            z0 = z0.reshape(_QL, _B, 4 * _H) + vz_s[...][None]
            whhT = whhT_s[...]
            h = ch_ref[...]
            c = cc_ref[...]
            carry_h = h
            carry_c = c
            for t in range(_QL):
                z = z0[t] + jnp.dot(h.astype(jnp.bfloat16), whhT,
                                    preferred_element_type=jnp.float32)
                zi = z[:, 0 * _H:1 * _H]
                zf = z[:, 1 * _H:2 * _H]
                zg = z[:, 2 * _H:3 * _H]
                zo = z[:, 3 * _H:4 * _H]
                c = jax.nn.sigmoid(zf) * c + jax.nn.sigmoid(zi) * jnp.tanh(zg)
                h = jax.nn.sigmoid(zo) * jnp.tanh(c)
                hs_ref[:, pl.ds(i * _QL + t, 1), :] = h.reshape(_B, 1, _H)
                s = sel_ref[0, :, t:t + 1]
                carry_h = s * h + (1.0 - s) * carry_h
                carry_c = s * c + (1.0 - s) * carry_c
            ch_ref[...] = carry_h
            cc_ref[...] = carry_c

        @pl.when(i == _MQ)
        def _tr():
            hs2 = hs_ref[...].reshape(_N, _H)
            hsT_s[...] = hs2.T.astype(jnp.bfloat16)

        @pl.when(i >= _MQ)
        def _proj():
            ow = ow_ref[...].astype(jnp.bfloat16)
            acc = jnp.dot(ow, hsT_s[...], preferred_element_type=jnp.float32)
            out_ref[...] = (acc + ob_ref[...]) * vm_ref[...]

    sc = _MQ - 1

    def _chunk_ix(i):
        return (jnp.minimum(i, sc), 0, 0)

    def _tile_ix(i):
        return (jnp.maximum(i - _MQ, 0), 0)

    return pl.pallas_call(
        body,
        grid=(_MQ + _NVT,),
        in_specs=[
            pl.BlockSpec((_QL, _B, _E), _chunk_ix),
            pl.BlockSpec((4 * _H, _E + _DV), lambda i: (0, 0)),
            pl.BlockSpec((4 * _H, _H), lambda i: (0, 0)),
            pl.BlockSpec((_VT, _H), _tile_ix),
            pl.BlockSpec((_DV, _B), lambda i: (0, 0)),
            pl.BlockSpec((1, 4 * _H), lambda i: (0, 0)),
            pl.BlockSpec((1, _B, _QL), _chunk_ix),
            pl.BlockSpec((1, _N), lambda i: (0, 0)),
            pl.BlockSpec((_VT, 1), _tile_ix),
        ],
        out_specs=pl.BlockSpec((_VT, _N), _tile_ix),
        out_shape=jax.ShapeDtypeStruct((_V, _N), jnp.float32),
        scratch_shapes=[
            pltpu.VMEM((_E, 4 * _H), jnp.bfloat16),
            pltpu.VMEM((_H, 4 * _H), jnp.bfloat16),
            pltpu.VMEM((_B, 4 * _H), jnp.float32),
            pltpu.VMEM((_B, _S, _H), jnp.float32),
            pltpu.VMEM((_H, _N), jnp.bfloat16),
            pltpu.VMEM((_B, _H), jnp.float32),
            pltpu.VMEM((_B, _H), jnp.float32),
        ],
    )(xe3, Wih, Whh, outW, visT, b2d, sel, vmaskr, outbc)


def kernel(source_questions, question_lengths, visual_features, unrolled_dialogue,
           cumulative_lengths, num_questions, object_categories, object_bboxes,
           emb, Wih, Whh, b, outW, outb, g_emb, g_Wih, g_Whh, g_b,
           cat_emb, W1, b1, W2, b2):
    toks = source_questions.transpose(1, 2, 0).reshape(_N).astype(jnp.int32)
    xe = _sc_gather(emb, toks)
    xe3 = xe.reshape(_S, _B, _E)

    lens = question_lengths.astype(jnp.int32)
    nq = num_questions.astype(jnp.int32)
    running = jnp.arange(_MQ)[None, :] < nq[:, None]
    tix = jnp.arange(_QL)
    sel = (lens[:, :, None] - 1 == tix[None, None, :]) & running[:, :, None]
    sel = sel.transpose(1, 0, 2).astype(jnp.float32)
    valid = (tix[None, None, :] < lens[:, :, None]) & running[:, :, None]
    vmaskr = valid.reshape(1, _N).astype(jnp.float32)

    outT = _scan_proj(xe3, Wih, Whh, outW, visual_features.T,
                      b.reshape(1, 4 * _H), sel, vmaskr,
                      outb.reshape(_V, 1))
    return outT.T


# speedup vs baseline: 1.0743x; 1.0038x over previous
import jax
import jax.numpy as jnp
from jax import lax
from jax.experimental import pallas as pl
from jax.experimental.pallas import tpu as pltpu
from jax.experimental.pallas import tpu_sc as plsc

_B, _MQ, _QL, _V, _E, _H, _DV = 16, 8, 16, 5000, 512, 512, 1024
_S = _MQ * _QL
_N = _S * _B
_VT = 1024
_NVT = (_V + _VT - 1) // _VT


def _sc_gather(table, idx):
    n, d = idx.shape[0], table.shape[1]
    nw = 32
    per = n // nw
    mesh = plsc.VectorSubcoreMesh(core_axis_name="c", subcore_axis_name="s")

    def body(table_hbm, idx_hbm, out_hbm, idx_v, rows_v, sem):
        wid = lax.axis_index("s") * 2 + lax.axis_index("c")
        base = wid * per
        pltpu.sync_copy(idx_hbm.at[pl.ds(base, per)], idx_v)
        pltpu.async_copy(table_hbm.at[idx_v], rows_v, sem).wait()
        pltpu.sync_copy(rows_v, out_hbm.at[pl.ds(base, per)])

    return pl.kernel(
        body,
        out_type=jax.ShapeDtypeStruct((n, d), table.dtype),
        mesh=mesh,
        scratch_types=[
            pltpu.VMEM((per,), jnp.int32),
            pltpu.VMEM((per, d), table.dtype),
            pltpu.SemaphoreType.DMA,
        ],
    )(table, idx)


def _scan_proj(xe3, Wih, Whh, outW, visT, b2d, sel, vmaskr, outbc):
    def body(xe_ref, wih_ref, whh_ref, ow_ref, vt_ref, b_ref, sel_ref,
             vm_ref, ob_ref, out_ref,
             wieT_s, whhT_s, vz_s, hc_ref, hs_ref, hsT_s, ch_ref, cc_ref):
        i = pl.program_id(0)

        @pl.when(i == 0)
        def _():
            wih = wih_ref[...]
            wieT_s[...] = wih[:, :_E].T.astype(jnp.bfloat16)
            whhT_s[...] = whh_ref[...].T.astype(jnp.bfloat16)
            viszT = jnp.dot(wih[:, _E:], vt_ref[...],
                            preferred_element_type=jnp.float32)
            vz_s[...] = viszT.T + b_ref[...]
            ch_ref[...] = jnp.zeros_like(ch_ref)
            cc_ref[...] = jnp.zeros_like(cc_ref)

        @pl.when(i < _MQ)
        def _scan():
            xe = xe_ref[...].reshape(_QL * _B, _E).astype(jnp.bfloat16)
            z0 = jnp.dot(xe, wieT_s[...], preferred_element_type=jnp.float32)
            z0 = z0.reshape(_QL, _B, 4 * _H) + vz_s[...][None]
            whhT = whhT_s[...]
            h = ch_ref[...]
            c = cc_ref[...]
            carry_h = h
            carry_c = c
            for t in range(_QL):
                z = z0[t] + jnp.dot(h.astype(jnp.bfloat16), whhT,
                                    preferred_element_type=jnp.float32)
                zi = z[:, 0 * _H:1 * _H]
                zf = z[:, 1 * _H:2 * _H]
                zg = z[:, 2 * _H:3 * _H]
                zo = z[:, 3 * _H:4 * _H]
                c = jax.nn.sigmoid(zf) * c + jax.nn.sigmoid(zi) * jnp.tanh(zg)
                h = jax.nn.sigmoid(zo) * jnp.tanh(c)
                hc_ref[:, t, :] = h
                s = sel_ref[0, :, t:t + 1]
                carry_h = s * h + (1.0 - s) * carry_h
                carry_c = s * c + (1.0 - s) * carry_c
            ch_ref[...] = carry_h
            cc_ref[...] = carry_c
            hs_ref[:, pl.ds(i * _QL, _QL), :] = hc_ref[...]

        @pl.when(i == _MQ)
        def _tr():
            hs2 = hs_ref[...].reshape(_N, _H)
            hsT_s[...] = hs2.T.astype(jnp.bfloat16)

        @pl.when(i >= _MQ)
        def _proj():
            ow = ow_ref[...].astype(jnp.bfloat16)
            acc = jnp.dot(ow, hsT_s[...], preferred_element_type=jnp.float32)
            out_ref[...] = (acc + ob_ref[...]) * vm_ref[...]

    sc = _MQ - 1

    def _chunk_ix(i):
        return (jnp.minimum(i, sc), 0, 0)

    def _tile_ix(i):
        return (jnp.maximum(i - _MQ, 0), 0)

    return pl.pallas_call(
        body,
        grid=(_MQ + _NVT,),
        in_specs=[
            pl.BlockSpec((_QL, _B, _E), _chunk_ix),
            pl.BlockSpec((4 * _H, _E + _DV), lambda i: (0, 0)),
            pl.BlockSpec((4 * _H, _H), lambda i: (0, 0)),
            pl.BlockSpec((_VT, _H), _tile_ix),
            pl.BlockSpec((_DV, _B), lambda i: (0, 0)),
            pl.BlockSpec((1, 4 * _H), lambda i: (0, 0)),
            pl.BlockSpec((1, _B, _QL), _chunk_ix),
            pl.BlockSpec((1, _N), lambda i: (0, 0)),
            pl.BlockSpec((_VT, 1), _tile_ix),
        ],
        out_specs=pl.BlockSpec((_VT, _N), _tile_ix),
        out_shape=jax.ShapeDtypeStruct((_V, _N), jnp.float32),
        scratch_shapes=[
            pltpu.VMEM((_E, 4 * _H), jnp.bfloat16),
            pltpu.VMEM((_H, 4 * _H), jnp.bfloat16),
            pltpu.VMEM((_B, 4 * _H), jnp.float32),
            pltpu.VMEM((_B, _QL, _H), jnp.float32),
            pltpu.VMEM((_B, _S, _H), jnp.float32),
            pltpu.VMEM((_H, _N), jnp.bfloat16),
            pltpu.VMEM((_B, _H), jnp.float32),
            pltpu.VMEM((_B, _H), jnp.float32),
        ],
    )(xe3, Wih, Whh, outW, visT, b2d, sel, vmaskr, outbc)


def kernel(source_questions, question_lengths, visual_features, unrolled_dialogue,
           cumulative_lengths, num_questions, object_categories, object_bboxes,
           emb, Wih, Whh, b, outW, outb, g_emb, g_Wih, g_Whh, g_b,
           cat_emb, W1, b1, W2, b2):
    toks = source_questions.transpose(1, 2, 0).reshape(_N).astype(jnp.int32)
    xe = _sc_gather(emb, toks)
    xe3 = xe.reshape(_S, _B, _E)

    lens = question_lengths.astype(jnp.int32)
    nq = num_questions.astype(jnp.int32)
    running = jnp.arange(_MQ)[None, :] < nq[:, None]
    tix = jnp.arange(_QL)
    sel = (lens[:, :, None] - 1 == tix[None, None, :]) & running[:, :, None]
    sel = sel.transpose(1, 0, 2).astype(jnp.float32)
    valid = (tix[None, None, :] < lens[:, :, None]) & running[:, :, None]
    vmaskr = valid.reshape(1, _N).astype(jnp.float32)

    outT = _scan_proj(xe3, Wih, Whh, outW, visual_features.T,
                      b.reshape(1, 4 * _H), sel, vmaskr,
                      outb.reshape(_V, 1))
    return outT.T
